# baseline (speedup 1.0000x reference)
import functools

import jax
import jax.numpy as jnp
from jax import lax
from jax.experimental import pallas as pl
from jax.experimental.pallas import tpu as pltpu
from jax.experimental.pallas import tpu_sc as plsc

_NC = 2
_NS = 16
_NW = _NC * _NS
_CHUNK = 128
_NPAD = 10240
_RPT = _NPAD // _NS

_f32 = jnp.float32


def _lrelu(v):
    return jnp.where(v > 0, v, 0.01 * v)


def _sc_mesh():
    return plsc.VectorSubcoreMesh(core_axis_name="c", subcore_axis_name="s")


_SC_PARAMS = pltpu.CompilerParams(use_tc_tiling_on_sc=False)


def _sc_gather(h, src_p, chunks_per_worker):
    e_pad = src_p.shape[0]

    def body(h_hbm, src_hbm, g_hbm, idx_v, rows_v, sem):
        c = lax.axis_index("c")
        s = lax.axis_index("s")
        base0 = (s * _NC + c) * (chunks_per_worker * _CHUNK)

        def step(i, carry):
            base = base0 + i * _CHUNK
            pltpu.sync_copy(src_hbm.at[pl.ds(base, _CHUNK)], idx_v)
            pltpu.async_copy(h_hbm.at[idx_v], rows_v, sem).wait()
            pltpu.sync_copy(rows_v, g_hbm.at[pl.ds(base, _CHUNK)])
            return carry

        lax.fori_loop(0, chunks_per_worker, step, 0)

    f = pl.kernel(
        body,
        out_type=jax.ShapeDtypeStruct((e_pad, 32), _f32),
        mesh=_sc_mesh(),
        compiler_params=_SC_PARAMS,
        scratch_types=[
            pltpu.VMEM((_CHUNK,), jnp.int32),
            pltpu.VMEM((_CHUNK, 32), _f32),
            pltpu.SemaphoreType.DMA,
        ],
    )
    return f(h, src_p)


def _sc_scatter(msg, dst_p, zrows, chunks_per_worker):
    def body(msg_hbm, dst_hbm, z_hbm, out_hbm, idx_v, rows_v, acc, sem):
        c = lax.axis_index("c")
        s = lax.axis_index("s")
        pltpu.sync_copy(z_hbm, acc.at[pl.ds(s * _RPT, _RPT)])
        plsc.subcore_barrier()
        base0 = (c * _NS + s) * (chunks_per_worker * _CHUNK)

        def step(i, carry):
            base = base0 + i * _CHUNK
            pltpu.sync_copy(dst_hbm.at[pl.ds(base, _CHUNK)], idx_v)
            pltpu.sync_copy(msg_hbm.at[pl.ds(base, _CHUNK)], rows_v)
            pltpu.async_copy(rows_v, acc.at[idx_v], sem, add=True).wait()
            return carry

        lax.fori_loop(0, chunks_per_worker, step, 0)
        plsc.subcore_barrier()
        pltpu.sync_copy(acc.at[pl.ds(s * _RPT, _RPT)],
                        out_hbm.at[pl.ds(c * _NPAD + s * _RPT, _RPT)])

    f = pl.kernel(
        body,
        out_type=jax.ShapeDtypeStruct((2 * _NPAD, 32), _f32),
        mesh=_sc_mesh(),
        compiler_params=_SC_PARAMS,
        scratch_types=[
            pltpu.VMEM((_CHUNK,), jnp.int32),
            pltpu.VMEM((_CHUNK, 32), _f32),
            pltpu.VMEM_SHARED((_NPAD, 32), _f32),
            pltpu.SemaphoreType.DMA,
        ],
    )
    return f(msg, dst_p, zrows)


def _sc_degree(dst_p, ones_p, zvec, chunks_per_worker):
    def body(dst_hbm, ones_hbm, z_hbm, out_hbm, idx_v, vals_v, acc, sem):
        c = lax.axis_index("c")
        s = lax.axis_index("s")
        pltpu.sync_copy(z_hbm, acc.at[pl.ds(s * _RPT, _RPT)])
        plsc.subcore_barrier()
        base0 = (c * _NS + s) * (chunks_per_worker * _CHUNK)

        def step(i, carry):
            base = base0 + i * _CHUNK
            pltpu.sync_copy(dst_hbm.at[pl.ds(base, _CHUNK)], idx_v)
            pltpu.sync_copy(ones_hbm.at[pl.ds(base, _CHUNK)], vals_v)
            pltpu.async_copy(vals_v, acc.at[idx_v], sem, add=True).wait()
            return carry

        lax.fori_loop(0, chunks_per_worker, step, 0)
        plsc.subcore_barrier()
        pltpu.sync_copy(acc.at[pl.ds(s * _RPT, _RPT)],
                        out_hbm.at[pl.ds(c * _NPAD + s * _RPT, _RPT)])

    f = pl.kernel(
        body,
        out_type=jax.ShapeDtypeStruct((2 * _NPAD,), _f32),
        mesh=_sc_mesh(),
        compiler_params=_SC_PARAMS,
        scratch_types=[
            pltpu.VMEM((_CHUNK,), jnp.int32),
            pltpu.VMEM((_CHUNK,), _f32),
            pltpu.VMEM_SHARED((_NPAD,), _f32),
            pltpu.SemaphoreType.DMA,
        ],
    )
    return f(dst_p, ones_p, zvec)


def _tc_prologue(x, lin0_Wt, lin0_b_row, deg0, deg1):
    n = x.shape[0]

    def body(x_ref, w_ref, b_ref, d0_ref, d1_ref, h_ref, rdeg_ref):
        h = jnp.dot(x_ref[...], w_ref[...], preferred_element_type=_f32)
        h_ref[...] = _lrelu(h + b_ref[...])
        deg = jnp.maximum(d0_ref[...] + d1_ref[...], 1.0)
        rdeg_ref[...] = 1.0 / deg

    return pl.pallas_call(
        body,
        out_shape=(jax.ShapeDtypeStruct((n, 32), _f32),
                   jax.ShapeDtypeStruct((n, 1), _f32)),
    )(x, lin0_Wt, lin0_b_row, deg0, deg1)


def _tc_messages(g, ea_p, net1_Wt, net1_b_row, net2_Wt, net2_b_row, blk=1024):
    e_pad = g.shape[0]
    grid = e_pad // blk

    def body(g_ref, ea_ref, w1_ref, b1_ref, w2_ref, b2_ref, msg_ref):
        ea = ea_ref[...]
        w1 = w1_ref[...]
        eh = b1_ref[...] + sum(
            ea[:, c:c + 1] * w1[c:c + 1, :] for c in range(4))
        eh = _lrelu(eh)
        w_blk = jnp.dot(eh, w2_ref[...], preferred_element_type=_f32)
        w_blk = w_blk + b2_ref[...]
        gb = g_ref[...]
        acc = jnp.zeros((blk, 32), _f32)
        for i in range(32):
            acc = acc + gb[:, i:i + 1] * w_blk[:, 32 * i:32 * i + 32]
        msg_ref[...] = acc

    return pl.pallas_call(
        body,
        grid=(grid,),
        in_specs=[
            pl.BlockSpec((blk, 32), lambda i: (i, 0)),
            pl.BlockSpec((blk, 4), lambda i: (i, 0)),
            pl.BlockSpec((4, 32), lambda i: (0, 0)),
            pl.BlockSpec((1, 32), lambda i: (0, 0)),
            pl.BlockSpec((32, 1024), lambda i: (0, 0)),
            pl.BlockSpec((1, 1024), lambda i: (0, 0)),
        ],
        out_specs=pl.BlockSpec((blk, 32), lambda i: (i, 0)),
        out_shape=jax.ShapeDtypeStruct((e_pad, 32), _f32),
    )(g, ea_p, net1_Wt, net1_b_row, net2_Wt, net2_b_row)


def _tc_gru(p0, p1, rdeg, h, root, conv_b_row, wih_t, whh_t, bih_row, bhh_row):
    n = h.shape[0]

    def body(p0_ref, p1_ref, rdeg_ref, h_ref, root_ref, cb_ref,
             wih_ref, whh_ref, bih_ref, bhh_ref, out_ref):
        h = h_ref[...]
        aggr = (p0_ref[...] + p1_ref[...]) * rdeg_ref[...]
        m = _lrelu(aggr + jnp.dot(h, root_ref[...],
                                  preferred_element_type=_f32) + cb_ref[...])
        gi = jnp.dot(m, wih_ref[...], preferred_element_type=_f32) + bih_ref[...]
        gh = jnp.dot(h, whh_ref[...], preferred_element_type=_f32) + bhh_ref[...]
        r = jax.nn.sigmoid(gi[:, 0:32] + gh[:, 0:32])
        z = jax.nn.sigmoid(gi[:, 32:64] + gh[:, 32:64])
        nn = jnp.tanh(gi[:, 64:96] + r * gh[:, 64:96])
        out_ref[...] = (1.0 - z) * nn + z * h

    return pl.pallas_call(
        body,
        out_shape=jax.ShapeDtypeStruct((n, 32), _f32),
    )(p0, p1, rdeg, h, root, conv_b_row, wih_t, whh_t, bih_row, bhh_row)


def _tc_set2set(h, batch_col, batch_row, lstm_bias_row, nb):
    n = h.shape[0]

    def body(h_ref, bc_ref, br_ref, lb_ref, out_ref):
        g = lb_ref[...]
        i_g = jax.nn.sigmoid(g[:, 0:32])
        f_g = jax.nn.sigmoid(g[:, 32:64])
        g_g = jnp.tanh(g[:, 64:96])
        o_g = jax.nn.sigmoid(g[:, 96:128])
        c_l = i_g * g_g
        qv = o_g * jnp.tanh(c_l)

        hv = h_ref[...]
        e_col = jnp.sum(hv * qv, axis=1, keepdims=True)
        iota_row = lax.broadcasted_iota(jnp.int32, (1, nb), 1)
        iota_col = lax.broadcasted_iota(jnp.int32, (nb, 1), 0)
        mnb_b = bc_ref[...] == iota_row
        mnb = mnb_b.astype(_f32)
        emax = jnp.max(jnp.where(mnb_b, e_col, -1e30), axis=0, keepdims=True)
        emax_g = jnp.sum(mnb * emax, axis=1, keepdims=True)
        ex = jnp.exp(e_col - emax_g)
        esum = jnp.sum(jnp.where(mnb_b, ex, 0.0), axis=0, keepdims=True)
        esum_g = jnp.sum(mnb * esum, axis=1, keepdims=True)
        a_col = ex / esum_g
        xw = a_col * hv
        mbn = (iota_col == br_ref[...]).astype(_f32)
        r_read = jnp.dot(mbn, xw, preferred_element_type=_f32)
        out_ref[...] = jnp.concatenate(
            [jnp.broadcast_to(qv, (nb, 32)), r_read], axis=1)

    return pl.pallas_call(
        body,
        out_shape=jax.ShapeDtypeStruct((nb, 64), _f32),
    )(h, batch_col, batch_row, lstm_bias_row)


def kernel(x, edge_index, edge_attr, batch, lin0_W, lin0_b, net1_W, net1_b,
           net2_W, net2_b, root, conv_b, gru_Wih, gru_Whh, gru_bih, gru_bhh,
           lstm_Wih, lstm_Whh, lstm_bih, lstm_bhh):
    n, _ = x.shape
    e = edge_index.shape[1]
    nb = lstm_Whh.shape[1] * 2

    cpw = -(-e // (_NW * _CHUNK))
    e_pad = _NW * _CHUNK * cpw
    pad = e_pad - e

    src_p = jnp.concatenate([edge_index[0], jnp.zeros((pad,), jnp.int32)])
    dst_p = jnp.concatenate([edge_index[1], jnp.full((pad,), n, jnp.int32)])
    ea_p = jnp.concatenate([edge_attr, jnp.zeros((pad, 4), _f32)], axis=0)
    ones_p = jnp.concatenate([jnp.ones((e,), _f32), jnp.zeros((pad,), _f32)])
    zrows = jnp.zeros((_RPT, 32), _f32)
    zvec = jnp.zeros((_RPT,), _f32)

    degp = _sc_degree(dst_p, ones_p, zvec, cpw)
    deg0 = degp[:n].reshape(n, 1)
    deg1 = degp[_NPAD:_NPAD + n].reshape(n, 1)

    h, rdeg = _tc_prologue(x, lin0_W.T, lin0_b.reshape(1, 32), deg0, deg1)

    net1_Wt = net1_W.T
    net1_b_row = net1_b.reshape(1, 32)
    net2_Wt = net2_W.T
    net2_b_row = net2_b.reshape(1, 1024)
    conv_b_row = conv_b.reshape(1, 32)
    wih_t = gru_Wih.T
    whh_t = gru_Whh.T
    bih_row = gru_bih.reshape(1, 96)
    bhh_row = gru_bhh.reshape(1, 96)

    for _ in range(6):
        g = _sc_gather(h, src_p, cpw)
        msg = _tc_messages(g, ea_p, net1_Wt, net1_b_row, net2_Wt, net2_b_row)
        parts = _sc_scatter(msg, dst_p, zrows, cpw)
        p0 = parts[:n]
        p1 = parts[_NPAD:_NPAD + n]
        h = _tc_gru(p0, p1, rdeg, h, root, conv_b_row, wih_t, whh_t,
                    bih_row, bhh_row)

    lstm_bias_row = (lstm_bih + lstm_bhh).reshape(1, 128)
    return _tc_set2set(h, batch.reshape(n, 1), batch.reshape(1, n),
                       lstm_bias_row, nb)


# speedup vs baseline: 2.9159x; 2.9159x over previous
import functools

import jax
import jax.numpy as jnp
from jax import lax
from jax.experimental import pallas as pl
from jax.experimental.pallas import tpu as pltpu
from jax.experimental.pallas import tpu_sc as plsc

_NC = 2
_NS = 16
_NW = _NC * _NS
_CHUNK = 128
_NPAD = 10240
_RPT = _NPAD // _NS

_f32 = jnp.float32


def _lrelu(v):
    return jnp.where(v > 0, v, 0.01 * v)


def _sc_mesh():
    return plsc.VectorSubcoreMesh(core_axis_name="c", subcore_axis_name="s")


_SC_PARAMS = pltpu.CompilerParams(use_tc_tiling_on_sc=False)


def _sc_gather(h, src_p, chunks_per_worker):
    e_pad = src_p.shape[0]

    def body(h_hbm, src_hbm, g_hbm, idx_v, rows_v, sem):
        c = lax.axis_index("c")
        s = lax.axis_index("s")
        base0 = (s * _NC + c) * (chunks_per_worker * _CHUNK)

        def step(i, carry):
            base = base0 + i * _CHUNK
            pltpu.sync_copy(src_hbm.at[pl.ds(base, _CHUNK)], idx_v)
            pltpu.async_copy(h_hbm.at[idx_v], rows_v, sem).wait()
            pltpu.sync_copy(rows_v, g_hbm.at[pl.ds(base, _CHUNK)])
            return carry

        lax.fori_loop(0, chunks_per_worker, step, 0)

    f = pl.kernel(
        body,
        out_type=jax.ShapeDtypeStruct((e_pad, 32), _f32),
        mesh=_sc_mesh(),
        compiler_params=_SC_PARAMS,
        scratch_types=[
            pltpu.VMEM((_CHUNK,), jnp.int32),
            pltpu.VMEM((_CHUNK, 32), _f32),
            pltpu.SemaphoreType.DMA,
        ],
    )
    return f(h, src_p)


def _sc_scatter(msg, dst_p, zrows, chunks_per_worker):
    def body(msg_hbm, dst_hbm, z_hbm, out_hbm, idx_v, rows_v, acc, sem):
        c = lax.axis_index("c")
        s = lax.axis_index("s")
        pltpu.sync_copy(z_hbm, acc.at[pl.ds(s * _RPT, _RPT)])
        plsc.subcore_barrier()
        base0 = (c * _NS + s) * (chunks_per_worker * _CHUNK)

        def step(i, carry):
            base = base0 + i * _CHUNK
            pltpu.sync_copy(dst_hbm.at[pl.ds(base, _CHUNK)], idx_v)
            pltpu.sync_copy(msg_hbm.at[pl.ds(base, _CHUNK)], rows_v)
            pltpu.async_copy(rows_v, acc.at[idx_v], sem, add=True).wait()
            return carry

        lax.fori_loop(0, chunks_per_worker, step, 0)
        plsc.subcore_barrier()
        pltpu.sync_copy(acc.at[pl.ds(s * _RPT, _RPT)],
                        out_hbm.at[pl.ds(c * _NPAD + s * _RPT, _RPT)])

    f = pl.kernel(
        body,
        out_type=jax.ShapeDtypeStruct((2 * _NPAD, 32), _f32),
        mesh=_sc_mesh(),
        compiler_params=_SC_PARAMS,
        scratch_types=[
            pltpu.VMEM((_CHUNK,), jnp.int32),
            pltpu.VMEM((_CHUNK, 32), _f32),
            pltpu.VMEM_SHARED((_NPAD, 32), _f32),
            pltpu.SemaphoreType.DMA,
        ],
    )
    return f(msg, dst_p, zrows)


def _sc_degree(dst_p, ones_p, zvec, chunks_per_worker):
    def body(dst_hbm, ones_hbm, z_hbm, out_hbm, idx_v, vals_v, acc, sem):
        c = lax.axis_index("c")
        s = lax.axis_index("s")
        pltpu.sync_copy(z_hbm, acc.at[pl.ds(s * _RPT, _RPT)])
        plsc.subcore_barrier()
        base0 = (c * _NS + s) * (chunks_per_worker * _CHUNK)

        def step(i, carry):
            base = base0 + i * _CHUNK
            pltpu.sync_copy(dst_hbm.at[pl.ds(base, _CHUNK)], idx_v)
            pltpu.sync_copy(ones_hbm.at[pl.ds(base, _CHUNK)], vals_v)
            pltpu.async_copy(vals_v, acc.at[idx_v], sem, add=True).wait()
            return carry

        lax.fori_loop(0, chunks_per_worker, step, 0)
        plsc.subcore_barrier()
        pltpu.sync_copy(acc.at[pl.ds(s * _RPT, _RPT)],
                        out_hbm.at[pl.ds(c * _NPAD + s * _RPT, _RPT)])

    f = pl.kernel(
        body,
        out_type=jax.ShapeDtypeStruct((2 * _NPAD,), _f32),
        mesh=_sc_mesh(),
        compiler_params=_SC_PARAMS,
        scratch_types=[
            pltpu.VMEM((_CHUNK,), jnp.int32),
            pltpu.VMEM((_CHUNK,), _f32),
            pltpu.VMEM_SHARED((_NPAD,), _f32),
            pltpu.SemaphoreType.DMA,
        ],
    )
    return f(dst_p, ones_p, zvec)


def _tc_prologue(x, lin0_Wt, lin0_b_row, deg0, deg1):
    n = x.shape[0]

    def body(x_ref, w_ref, b_ref, d0_ref, d1_ref, h_ref, rdeg_ref):
        h = jnp.dot(x_ref[...], w_ref[...], preferred_element_type=_f32)
        h_ref[...] = _lrelu(h + b_ref[...])
        deg = jnp.maximum(d0_ref[...] + d1_ref[...], 1.0)
        rdeg_ref[...] = 1.0 / deg

    return pl.pallas_call(
        body,
        out_shape=(jax.ShapeDtypeStruct((n, 32), _f32),
                   jax.ShapeDtypeStruct((n, 1), _f32)),
    )(x, lin0_Wt, lin0_b_row, deg0, deg1)


def _tc_messages(g, ea_aug_t, net1_aug, net2_aug, blk=512):
    e_pad = g.shape[0]
    grid = e_pad // blk

    def body(g_ref, ea_ref, w1_ref, w2_ref, msg_ref):
        eh = _lrelu(jnp.dot(w1_ref[...], ea_ref[...],
                            preferred_element_type=_f32))
        w_t = jnp.dot(w2_ref[...], eh,
                      preferred_element_type=_f32)
        gt = g_ref[...].T
        acc = jnp.zeros((32, blk), _f32)
        for i in range(32):
            acc = acc + gt[i:i + 1, :] * w_t[32 * i:32 * i + 32, :]
        msg_ref[...] = acc.T

    return pl.pallas_call(
        body,
        grid=(grid,),
        in_specs=[
            pl.BlockSpec((blk, 32), lambda i: (i, 0)),
            pl.BlockSpec((5, blk), lambda i: (0, i)),
            pl.BlockSpec((33, 5), lambda i: (0, 0)),
            pl.BlockSpec((1024, 33), lambda i: (0, 0)),
        ],
        out_specs=pl.BlockSpec((blk, 32), lambda i: (i, 0)),
        out_shape=jax.ShapeDtypeStruct((e_pad, 32), _f32),
    )(g, ea_aug_t, net1_aug, net2_aug)


def _tc_gru(p0, p1, rdeg, h, root, conv_b_row, wih_t, whh_t, bih_row, bhh_row):
    n = h.shape[0]

    def body(p0_ref, p1_ref, rdeg_ref, h_ref, root_ref, cb_ref,
             wih_ref, whh_ref, bih_ref, bhh_ref, out_ref):
        h = h_ref[...]
        aggr = (p0_ref[...] + p1_ref[...]) * rdeg_ref[...]
        m = _lrelu(aggr + jnp.dot(h, root_ref[...],
                                  preferred_element_type=_f32) + cb_ref[...])
        gi = jnp.dot(m, wih_ref[...], preferred_element_type=_f32) + bih_ref[...]
        gh = jnp.dot(h, whh_ref[...], preferred_element_type=_f32) + bhh_ref[...]
        r = jax.nn.sigmoid(gi[:, 0:32] + gh[:, 0:32])
        z = jax.nn.sigmoid(gi[:, 32:64] + gh[:, 32:64])
        nn = jnp.tanh(gi[:, 64:96] + r * gh[:, 64:96])
        out_ref[...] = (1.0 - z) * nn + z * h

    return pl.pallas_call(
        body,
        out_shape=jax.ShapeDtypeStruct((n, 32), _f32),
    )(p0, p1, rdeg, h, root, conv_b_row, wih_t, whh_t, bih_row, bhh_row)


def _tc_set2set(h, batch_col, batch_row, lstm_bias_row, nb):
    n = h.shape[0]

    def body(h_ref, bc_ref, br_ref, lb_ref, out_ref):
        g = lb_ref[...]
        i_g = jax.nn.sigmoid(g[:, 0:32])
        f_g = jax.nn.sigmoid(g[:, 32:64])
        g_g = jnp.tanh(g[:, 64:96])
        o_g = jax.nn.sigmoid(g[:, 96:128])
        c_l = i_g * g_g
        qv = o_g * jnp.tanh(c_l)

        hv = h_ref[...]
        e_col = jnp.sum(hv * qv, axis=1, keepdims=True)
        iota_row = lax.broadcasted_iota(jnp.int32, (1, nb), 1)
        iota_col = lax.broadcasted_iota(jnp.int32, (nb, 1), 0)
        mnb_b = bc_ref[...] == iota_row
        mnb = mnb_b.astype(_f32)
        emax = jnp.max(jnp.where(mnb_b, e_col, -1e30), axis=0, keepdims=True)
        emax_g = jnp.sum(mnb * emax, axis=1, keepdims=True)
        ex = jnp.exp(e_col - emax_g)
        esum = jnp.sum(jnp.where(mnb_b, ex, 0.0), axis=0, keepdims=True)
        esum_g = jnp.sum(mnb * esum, axis=1, keepdims=True)
        a_col = ex / esum_g
        xw = a_col * hv
        mbn = (iota_col == br_ref[...]).astype(_f32)
        r_read = jnp.dot(mbn, xw, preferred_element_type=_f32)
        out_ref[...] = jnp.concatenate(
            [jnp.broadcast_to(qv, (nb, 32)), r_read], axis=1)

    return pl.pallas_call(
        body,
        out_shape=jax.ShapeDtypeStruct((nb, 64), _f32),
    )(h, batch_col, batch_row, lstm_bias_row)


def kernel(x, edge_index, edge_attr, batch, lin0_W, lin0_b, net1_W, net1_b,
           net2_W, net2_b, root, conv_b, gru_Wih, gru_Whh, gru_bih, gru_bhh,
           lstm_Wih, lstm_Whh, lstm_bih, lstm_bhh):
    n, _ = x.shape
    e = edge_index.shape[1]
    nb = lstm_Whh.shape[1] * 2

    cpw = -(-e // (_NW * _CHUNK))
    e_pad = _NW * _CHUNK * cpw
    pad = e_pad - e

    src_p = jnp.concatenate([edge_index[0], jnp.zeros((pad,), jnp.int32)])
    dst_p = jnp.concatenate([edge_index[1], jnp.full((pad,), n, jnp.int32)])
    ea_p = jnp.concatenate([edge_attr, jnp.zeros((pad, 4), _f32)], axis=0)
    ones_p = jnp.concatenate([jnp.ones((e,), _f32), jnp.zeros((pad,), _f32)])
    zrows = jnp.zeros((_RPT, 32), _f32)
    zvec = jnp.zeros((_RPT,), _f32)

    degp = _sc_degree(dst_p, ones_p, zvec, cpw)
    deg0 = degp[:n].reshape(n, 1)
    deg1 = degp[_NPAD:_NPAD + n].reshape(n, 1)

    h, rdeg = _tc_prologue(x, lin0_W.T, lin0_b.reshape(1, 32), deg0, deg1)

    ea_aug_t = jnp.concatenate([ea_p.T, jnp.ones((1, e_pad), _f32)], axis=0)
    net1_aug = jnp.concatenate([
        jnp.concatenate([net1_W, net1_b.reshape(32, 1)], axis=1),
        jnp.array([[0.0, 0.0, 0.0, 0.0, 1.0]], _f32)], axis=0)
    net2_aug = jnp.concatenate([net2_W, net2_b.reshape(1024, 1)], axis=1)
    conv_b_row = conv_b.reshape(1, 32)
    wih_t = gru_Wih.T
    whh_t = gru_Whh.T
    bih_row = gru_bih.reshape(1, 96)
    bhh_row = gru_bhh.reshape(1, 96)

    for _ in range(6):
        g = _sc_gather(h, src_p, cpw)
        msg = _tc_messages(g, ea_aug_t, net1_aug, net2_aug)
        parts = _sc_scatter(msg, dst_p, zrows, cpw)
        p0 = parts[:n]
        p1 = parts[_NPAD:_NPAD + n]
        h = _tc_gru(p0, p1, rdeg, h, root, conv_b_row, wih_t, whh_t,
                    bih_row, bhh_row)

    lstm_bias_row = (lstm_bih + lstm_bhh).reshape(1, 128)
    return _tc_set2set(h, batch.reshape(n, 1), batch.reshape(1, n),
                       lstm_bias_row, nb)


# speedup vs baseline: 3.4954x; 1.1987x over previous
import functools

import jax
import jax.numpy as jnp
from jax import lax
from jax.experimental import pallas as pl
from jax.experimental.pallas import tpu as pltpu
from jax.experimental.pallas import tpu_sc as plsc

_NC = 2
_NS = 16
_NW = _NC * _NS
_CHUNK = 128
_NPAD = 10240
_RPT = _NPAD // _NS

_f32 = jnp.float32


def _lrelu(v):
    return jnp.where(v > 0, v, 0.01 * v)


def _sc_mesh():
    return plsc.VectorSubcoreMesh(core_axis_name="c", subcore_axis_name="s")


_SC_PARAMS = pltpu.CompilerParams(use_tc_tiling_on_sc=False)


def _sc_gather(h, src_p, chunks_per_worker):
    e_pad = src_p.shape[0]

    def body(h_hbm, src_hbm, g_hbm, idx_v, rows_v, sem):
        c = lax.axis_index("c")
        s = lax.axis_index("s")
        base0 = (s * _NC + c) * (chunks_per_worker * _CHUNK)

        def step(i, carry):
            base = base0 + i * _CHUNK
            pltpu.sync_copy(src_hbm.at[pl.ds(base, _CHUNK)], idx_v)
            pltpu.async_copy(h_hbm.at[idx_v], rows_v, sem).wait()
            pltpu.sync_copy(rows_v, g_hbm.at[pl.ds(base, _CHUNK)])
            return carry

        lax.fori_loop(0, chunks_per_worker, step, 0)

    f = pl.kernel(
        body,
        out_type=jax.ShapeDtypeStruct((e_pad, 32), _f32),
        mesh=_sc_mesh(),
        compiler_params=_SC_PARAMS,
        scratch_types=[
            pltpu.VMEM((_CHUNK,), jnp.int32),
            pltpu.VMEM((_CHUNK, 32), _f32),
            pltpu.SemaphoreType.DMA,
        ],
    )
    return f(h, src_p)


def _sc_scatter(msg, dst_p, zrows, chunks_per_worker):
    def body(msg_hbm, dst_hbm, z_hbm, out_hbm, idx_v, rows_v, acc, sem):
        c = lax.axis_index("c")
        s = lax.axis_index("s")
        pltpu.sync_copy(z_hbm, acc.at[pl.ds(s * _RPT, _RPT)])
        plsc.subcore_barrier()
        base0 = (c * _NS + s) * (chunks_per_worker * _CHUNK)

        def step(i, carry):
            base = base0 + i * _CHUNK
            pltpu.sync_copy(dst_hbm.at[pl.ds(base, _CHUNK)], idx_v)
            pltpu.sync_copy(msg_hbm.at[pl.ds(base, _CHUNK)], rows_v)
            pltpu.async_copy(rows_v, acc.at[idx_v], sem, add=True).wait()
            return carry

        lax.fori_loop(0, chunks_per_worker, step, 0)
        plsc.subcore_barrier()
        pltpu.sync_copy(acc.at[pl.ds(s * _RPT, _RPT)],
                        out_hbm.at[pl.ds(c * _NPAD + s * _RPT, _RPT)])

    f = pl.kernel(
        body,
        out_type=jax.ShapeDtypeStruct((2 * _NPAD, 32), _f32),
        mesh=_sc_mesh(),
        compiler_params=_SC_PARAMS,
        scratch_types=[
            pltpu.VMEM((_CHUNK,), jnp.int32),
            pltpu.VMEM((_CHUNK, 32), _f32),
            pltpu.VMEM_SHARED((_NPAD, 32), _f32),
            pltpu.SemaphoreType.DMA,
        ],
    )
    return f(msg, dst_p, zrows)


def _sc_degree(dst_p, ones_p, zvec, chunks_per_worker):
    def body(dst_hbm, ones_hbm, z_hbm, out_hbm, idx_v, vals_v, acc, sem):
        c = lax.axis_index("c")
        s = lax.axis_index("s")
        pltpu.sync_copy(z_hbm, acc.at[pl.ds(s * _RPT, _RPT)])
        plsc.subcore_barrier()
        base0 = (c * _NS + s) * (chunks_per_worker * _CHUNK)

        def step(i, carry):
            base = base0 + i * _CHUNK
            pltpu.sync_copy(dst_hbm.at[pl.ds(base, _CHUNK)], idx_v)
            pltpu.sync_copy(ones_hbm.at[pl.ds(base, _CHUNK)], vals_v)
            pltpu.async_copy(vals_v, acc.at[idx_v], sem, add=True).wait()
            return carry

        lax.fori_loop(0, chunks_per_worker, step, 0)
        plsc.subcore_barrier()
        pltpu.sync_copy(acc.at[pl.ds(s * _RPT, _RPT)],
                        out_hbm.at[pl.ds(c * _NPAD + s * _RPT, _RPT)])

    f = pl.kernel(
        body,
        out_type=jax.ShapeDtypeStruct((2 * _NPAD,), _f32),
        mesh=_sc_mesh(),
        compiler_params=_SC_PARAMS,
        scratch_types=[
            pltpu.VMEM((_CHUNK,), jnp.int32),
            pltpu.VMEM((_CHUNK,), _f32),
            pltpu.VMEM_SHARED((_NPAD,), _f32),
            pltpu.SemaphoreType.DMA,
        ],
    )
    return f(dst_p, ones_p, zvec)


def _tc_prologue(x, lin0_Wt, lin0_b_row, deg0, deg1):
    n = x.shape[0]

    def body(x_ref, w_ref, b_ref, d0_ref, d1_ref, h_ref, rdeg_ref):
        h = jnp.dot(x_ref[...], w_ref[...], preferred_element_type=_f32)
        h_ref[...] = _lrelu(h + b_ref[...])
        deg = jnp.maximum(d0_ref[...] + d1_ref[...], 1.0)
        rdeg_ref[...] = 1.0 / deg

    return pl.pallas_call(
        body,
        out_shape=(jax.ShapeDtypeStruct((n, 32), _f32),
                   jax.ShapeDtypeStruct((n, 1), _f32)),
    )(x, lin0_Wt, lin0_b_row, deg0, deg1)


def _tc_edge_act(ea_aug_t, net1_aug, blk=2048):
    e_pad = ea_aug_t.shape[1]
    grid = e_pad // blk

    def body(ea_ref, w1_ref, out_ref):
        out_ref[...] = _lrelu(jnp.dot(w1_ref[...], ea_ref[...],
                                      preferred_element_type=_f32))

    return pl.pallas_call(
        body,
        grid=(grid,),
        in_specs=[
            pl.BlockSpec((5, blk), lambda i: (0, i)),
            pl.BlockSpec((33, 5), lambda i: (0, 0)),
        ],
        out_specs=pl.BlockSpec((33, blk), lambda i: (0, i)),
        out_shape=jax.ShapeDtypeStruct((33, e_pad), _f32),
    )(ea_aug_t, net1_aug)


def _tc_messages(g, e_act, net2_aug, blk=1024):
    e_pad = g.shape[0]
    grid = e_pad // blk

    def body(g_ref, ea_ref, w2_ref, msg_ref):
        w_t = jnp.dot(w2_ref[...], ea_ref[...],
                      preferred_element_type=_f32)
        gt = g_ref[...].T
        accs = [jnp.zeros((32, blk), _f32) for _ in range(4)]
        for i in range(32):
            accs[i % 4] = accs[i % 4] + (
                gt[i:i + 1, :] * w_t[32 * i:32 * i + 32, :])
        msg_ref[...] = ((accs[0] + accs[1]) + (accs[2] + accs[3])).T

    return pl.pallas_call(
        body,
        grid=(grid,),
        in_specs=[
            pl.BlockSpec((blk, 32), lambda i: (i, 0)),
            pl.BlockSpec((33, blk), lambda i: (0, i)),
            pl.BlockSpec((1024, 33), lambda i: (0, 0)),
        ],
        out_specs=pl.BlockSpec((blk, 32), lambda i: (i, 0)),
        out_shape=jax.ShapeDtypeStruct((e_pad, 32), _f32),
    )(g, e_act, net2_aug)


def _tc_gru(p0, p1, rdeg, h, root, conv_b_row, wih_t, whh_t, bih_row, bhh_row):
    n = h.shape[0]

    def body(p0_ref, p1_ref, rdeg_ref, h_ref, root_ref, cb_ref,
             wih_ref, whh_ref, bih_ref, bhh_ref, out_ref):
        h = h_ref[...]
        aggr = (p0_ref[...] + p1_ref[...]) * rdeg_ref[...]
        m = _lrelu(aggr + jnp.dot(h, root_ref[...],
                                  preferred_element_type=_f32) + cb_ref[...])
        gi = jnp.dot(m, wih_ref[...], preferred_element_type=_f32) + bih_ref[...]
        gh = jnp.dot(h, whh_ref[...], preferred_element_type=_f32) + bhh_ref[...]
        r = jax.nn.sigmoid(gi[:, 0:32] + gh[:, 0:32])
        z = jax.nn.sigmoid(gi[:, 32:64] + gh[:, 32:64])
        nn = jnp.tanh(gi[:, 64:96] + r * gh[:, 64:96])
        out_ref[...] = (1.0 - z) * nn + z * h

    return pl.pallas_call(
        body,
        out_shape=jax.ShapeDtypeStruct((n, 32), _f32),
    )(p0, p1, rdeg, h, root, conv_b_row, wih_t, whh_t, bih_row, bhh_row)


def _tc_set2set(h, batch_col, batch_row, lstm_bias_row, nb):
    n = h.shape[0]

    def body(h_ref, bc_ref, br_ref, lb_ref, out_ref):
        g = lb_ref[...]
        i_g = jax.nn.sigmoid(g[:, 0:32])
        f_g = jax.nn.sigmoid(g[:, 32:64])
        g_g = jnp.tanh(g[:, 64:96])
        o_g = jax.nn.sigmoid(g[:, 96:128])
        c_l = i_g * g_g
        qv = o_g * jnp.tanh(c_l)

        hv = h_ref[...]
        e_col = jnp.sum(hv * qv, axis=1, keepdims=True)
        iota_row = lax.broadcasted_iota(jnp.int32, (1, nb), 1)
        iota_col = lax.broadcasted_iota(jnp.int32, (nb, 1), 0)
        mnb_b = bc_ref[...] == iota_row
        mnb = mnb_b.astype(_f32)
        emax = jnp.max(jnp.where(mnb_b, e_col, -1e30), axis=0, keepdims=True)
        emax_g = jnp.sum(mnb * emax, axis=1, keepdims=True)
        ex = jnp.exp(e_col - emax_g)
        esum = jnp.sum(jnp.where(mnb_b, ex, 0.0), axis=0, keepdims=True)
        esum_g = jnp.sum(mnb * esum, axis=1, keepdims=True)
        a_col = ex / esum_g
        xw = a_col * hv
        mbn = (iota_col == br_ref[...]).astype(_f32)
        r_read = jnp.dot(mbn, xw, preferred_element_type=_f32)
        out_ref[...] = jnp.concatenate(
            [jnp.broadcast_to(qv, (nb, 32)), r_read], axis=1)

    return pl.pallas_call(
        body,
        out_shape=jax.ShapeDtypeStruct((nb, 64), _f32),
    )(h, batch_col, batch_row, lstm_bias_row)


def kernel(x, edge_index, edge_attr, batch, lin0_W, lin0_b, net1_W, net1_b,
           net2_W, net2_b, root, conv_b, gru_Wih, gru_Whh, gru_bih, gru_bhh,
           lstm_Wih, lstm_Whh, lstm_bih, lstm_bhh):
    n, _ = x.shape
    e = edge_index.shape[1]
    nb = lstm_Whh.shape[1] * 2

    cpw = -(-e // (_NW * _CHUNK))
    e_pad = _NW * _CHUNK * cpw
    pad = e_pad - e

    src_p = jnp.concatenate([edge_index[0], jnp.zeros((pad,), jnp.int32)])
    dst_p = jnp.concatenate([edge_index[1], jnp.full((pad,), n, jnp.int32)])
    ea_p = jnp.concatenate([edge_attr, jnp.zeros((pad, 4), _f32)], axis=0)
    ones_p = jnp.concatenate([jnp.ones((e,), _f32), jnp.zeros((pad,), _f32)])
    zrows = jnp.zeros((_RPT, 32), _f32)
    zvec = jnp.zeros((_RPT,), _f32)

    degp = _sc_degree(dst_p, ones_p, zvec, cpw)
    deg0 = degp[:n].reshape(n, 1)
    deg1 = degp[_NPAD:_NPAD + n].reshape(n, 1)

    h, rdeg = _tc_prologue(x, lin0_W.T, lin0_b.reshape(1, 32), deg0, deg1)

    ea_aug_t = jnp.concatenate([ea_p.T, jnp.ones((1, e_pad), _f32)], axis=0)
    net1_aug = jnp.concatenate([
        jnp.concatenate([net1_W, net1_b.reshape(32, 1)], axis=1),
        jnp.array([[0.0, 0.0, 0.0, 0.0, 1.0]], _f32)], axis=0)
    net2_aug = jnp.concatenate([net2_W, net2_b.reshape(1024, 1)], axis=1)
    conv_b_row = conv_b.reshape(1, 32)
    wih_t = gru_Wih.T
    whh_t = gru_Whh.T
    bih_row = gru_bih.reshape(1, 96)
    bhh_row = gru_bhh.reshape(1, 96)

    e_act = _tc_edge_act(ea_aug_t, net1_aug)

    for _ in range(6):
        g = _sc_gather(h, src_p, cpw)
        msg = _tc_messages(g, e_act, net2_aug)
        parts = _sc_scatter(msg, dst_p, zrows, cpw)
        p0 = parts[:n]
        p1 = parts[_NPAD:_NPAD + n]
        h = _tc_gru(p0, p1, rdeg, h, root, conv_b_row, wih_t, whh_t,
                    bih_row, bhh_row)

    lstm_bias_row = (lstm_bih + lstm_bhh).reshape(1, 128)
    return _tc_set2set(h, batch.reshape(n, 1), batch.reshape(1, n),
                       lstm_bias_row, nb)


# speedup vs baseline: 3.5024x; 1.0020x over previous
import functools

import jax
import jax.numpy as jnp
from jax import lax
from jax.experimental import pallas as pl
from jax.experimental.pallas import tpu as pltpu
from jax.experimental.pallas import tpu_sc as plsc

_NC = 2
_NS = 16
_NW = _NC * _NS
_CHUNK = 128
_NPAD = 10240
_RPT = _NPAD // _NS

_f32 = jnp.float32


def _lrelu(v):
    return jnp.where(v > 0, v, 0.01 * v)


def _sc_mesh():
    return plsc.VectorSubcoreMesh(core_axis_name="c", subcore_axis_name="s")


_SC_PARAMS = pltpu.CompilerParams(use_tc_tiling_on_sc=False)


def _sc_gather(h, src_p, chunks_per_worker):
    e_pad = src_p.shape[0]

    def body(h_hbm, src_hbm, g_hbm, idx_v, rows_v, sem):
        c = lax.axis_index("c")
        s = lax.axis_index("s")
        base0 = (s * _NC + c) * (chunks_per_worker * _CHUNK)

        def step(i, carry):
            base = base0 + i * _CHUNK
            pltpu.sync_copy(src_hbm.at[pl.ds(base, _CHUNK)], idx_v)
            pltpu.async_copy(h_hbm.at[idx_v], rows_v, sem).wait()
            pltpu.sync_copy(rows_v, g_hbm.at[pl.ds(base, _CHUNK)])
            return carry

        lax.fori_loop(0, chunks_per_worker, step, 0)

    f = pl.kernel(
        body,
        out_type=jax.ShapeDtypeStruct((e_pad, 32), _f32),
        mesh=_sc_mesh(),
        compiler_params=_SC_PARAMS,
        scratch_types=[
            pltpu.VMEM((_CHUNK,), jnp.int32),
            pltpu.VMEM((_CHUNK, 32), _f32),
            pltpu.SemaphoreType.DMA,
        ],
    )
    return f(h, src_p)


def _sc_scatter(msg, dst_p, zrows, chunks_per_worker):
    def body(msg_hbm, dst_hbm, z_hbm, out_hbm, idx_v, rows_v, acc, sem):
        c = lax.axis_index("c")
        s = lax.axis_index("s")
        pltpu.sync_copy(z_hbm, acc.at[pl.ds(s * _RPT, _RPT)])
        plsc.subcore_barrier()
        base0 = (c * _NS + s) * (chunks_per_worker * _CHUNK)

        def step(i, carry):
            base = base0 + i * _CHUNK
            pltpu.sync_copy(dst_hbm.at[pl.ds(base, _CHUNK)], idx_v)
            pltpu.sync_copy(msg_hbm.at[pl.ds(base, _CHUNK)], rows_v)
            pltpu.async_copy(rows_v, acc.at[idx_v], sem, add=True).wait()
            return carry

        lax.fori_loop(0, chunks_per_worker, step, 0)
        plsc.subcore_barrier()
        pltpu.sync_copy(acc.at[pl.ds(s * _RPT, _RPT)],
                        out_hbm.at[pl.ds(c * _NPAD + s * _RPT, _RPT)])

    f = pl.kernel(
        body,
        out_type=jax.ShapeDtypeStruct((2 * _NPAD, 32), _f32),
        mesh=_sc_mesh(),
        compiler_params=_SC_PARAMS,
        scratch_types=[
            pltpu.VMEM((_CHUNK,), jnp.int32),
            pltpu.VMEM((_CHUNK, 32), _f32),
            pltpu.VMEM_SHARED((_NPAD, 32), _f32),
            pltpu.SemaphoreType.DMA,
        ],
    )
    return f(msg, dst_p, zrows)


def _sc_degree(dst_p, ones_p, zvec, chunks_per_worker):
    def body(dst_hbm, ones_hbm, z_hbm, out_hbm, idx_v, vals_v, acc, sem):
        c = lax.axis_index("c")
        s = lax.axis_index("s")
        pltpu.sync_copy(z_hbm, acc.at[pl.ds(s * _RPT, _RPT)])
        plsc.subcore_barrier()
        base0 = (c * _NS + s) * (chunks_per_worker * _CHUNK)

        def step(i, carry):
            base = base0 + i * _CHUNK
            pltpu.sync_copy(dst_hbm.at[pl.ds(base, _CHUNK)], idx_v)
            pltpu.sync_copy(ones_hbm.at[pl.ds(base, _CHUNK)], vals_v)
            pltpu.async_copy(vals_v, acc.at[idx_v], sem, add=True).wait()
            return carry

        lax.fori_loop(0, chunks_per_worker, step, 0)
        plsc.subcore_barrier()
        pltpu.sync_copy(acc.at[pl.ds(s * _RPT, _RPT)],
                        out_hbm.at[pl.ds(c * _NPAD + s * _RPT, _RPT)])

    f = pl.kernel(
        body,
        out_type=jax.ShapeDtypeStruct((2 * _NPAD,), _f32),
        mesh=_sc_mesh(),
        compiler_params=_SC_PARAMS,
        scratch_types=[
            pltpu.VMEM((_CHUNK,), jnp.int32),
            pltpu.VMEM((_CHUNK,), _f32),
            pltpu.VMEM_SHARED((_NPAD,), _f32),
            pltpu.SemaphoreType.DMA,
        ],
    )
    return f(dst_p, ones_p, zvec)


def _tc_prologue(x, lin0_Wt, lin0_b_row, deg0, deg1):
    n = x.shape[0]

    def body(x_ref, w_ref, b_ref, d0_ref, d1_ref, h_ref, rdeg_ref):
        h = jnp.dot(x_ref[...], w_ref[...], preferred_element_type=_f32)
        h_ref[...] = _lrelu(h + b_ref[...])
        deg = jnp.maximum(d0_ref[...] + d1_ref[...], 1.0)
        rdeg_ref[...] = 1.0 / deg

    return pl.pallas_call(
        body,
        out_shape=(jax.ShapeDtypeStruct((n, 32), _f32),
                   jax.ShapeDtypeStruct((n, 1), _f32)),
    )(x, lin0_Wt, lin0_b_row, deg0, deg1)


def _tc_edge_act(ea_aug_t, net1_aug, blk=2048):
    e_pad = ea_aug_t.shape[1]
    grid = e_pad // blk

    def body(ea_ref, w1_ref, out_ref):
        out_ref[...] = _lrelu(jnp.dot(w1_ref[...], ea_ref[...],
                                      preferred_element_type=_f32))

    return pl.pallas_call(
        body,
        grid=(grid,),
        in_specs=[
            pl.BlockSpec((5, blk), lambda i: (0, i)),
            pl.BlockSpec((33, 5), lambda i: (0, 0)),
        ],
        out_specs=pl.BlockSpec((33, blk), lambda i: (0, i)),
        out_shape=jax.ShapeDtypeStruct((33, e_pad), _f32),
    )(ea_aug_t, net1_aug)


def _tc_messages(g, e_act, net2_aug, blk=1024):
    e_pad = g.shape[0]
    grid = e_pad // blk

    def body(g_ref, ea_ref, w2_ref, msg_ref):
        w_t = jnp.dot(w2_ref[...].astype(jnp.bfloat16),
                      ea_ref[...].astype(jnp.bfloat16),
                      preferred_element_type=_f32)
        gt = g_ref[...].T
        accs = [jnp.zeros((32, blk), _f32) for _ in range(4)]
        for i in range(32):
            accs[i % 4] = accs[i % 4] + (
                gt[i:i + 1, :] * w_t[32 * i:32 * i + 32, :])
        msg_ref[...] = ((accs[0] + accs[1]) + (accs[2] + accs[3])).T

    return pl.pallas_call(
        body,
        grid=(grid,),
        in_specs=[
            pl.BlockSpec((blk, 32), lambda i: (i, 0)),
            pl.BlockSpec((33, blk), lambda i: (0, i)),
            pl.BlockSpec((1024, 33), lambda i: (0, 0)),
        ],
        out_specs=pl.BlockSpec((blk, 32), lambda i: (i, 0)),
        out_shape=jax.ShapeDtypeStruct((e_pad, 32), _f32),
    )(g, e_act, net2_aug)


def _tc_gru(p0, p1, rdeg, h, root, conv_b_row, wih_t, whh_t, bih_row, bhh_row):
    n = h.shape[0]

    def body(p0_ref, p1_ref, rdeg_ref, h_ref, root_ref, cb_ref,
             wih_ref, whh_ref, bih_ref, bhh_ref, out_ref):
        h = h_ref[...]
        aggr = (p0_ref[...] + p1_ref[...]) * rdeg_ref[...]
        m = _lrelu(aggr + jnp.dot(h, root_ref[...],
                                  preferred_element_type=_f32) + cb_ref[...])
        gi = jnp.dot(m, wih_ref[...], preferred_element_type=_f32) + bih_ref[...]
        gh = jnp.dot(h, whh_ref[...], preferred_element_type=_f32) + bhh_ref[...]
        r = jax.nn.sigmoid(gi[:, 0:32] + gh[:, 0:32])
        z = jax.nn.sigmoid(gi[:, 32:64] + gh[:, 32:64])
        nn = jnp.tanh(gi[:, 64:96] + r * gh[:, 64:96])
        out_ref[...] = (1.0 - z) * nn + z * h

    return pl.pallas_call(
        body,
        out_shape=jax.ShapeDtypeStruct((n, 32), _f32),
    )(p0, p1, rdeg, h, root, conv_b_row, wih_t, whh_t, bih_row, bhh_row)


def _tc_set2set(h, batch_col, batch_row, lstm_bias_row, nb):
    n = h.shape[0]

    def body(h_ref, bc_ref, br_ref, lb_ref, out_ref):
        g = lb_ref[...]
        i_g = jax.nn.sigmoid(g[:, 0:32])
        f_g = jax.nn.sigmoid(g[:, 32:64])
        g_g = jnp.tanh(g[:, 64:96])
        o_g = jax.nn.sigmoid(g[:, 96:128])
        c_l = i_g * g_g
        qv = o_g * jnp.tanh(c_l)

        hv = h_ref[...]
        e_col = jnp.sum(hv * qv, axis=1, keepdims=True)
        iota_row = lax.broadcasted_iota(jnp.int32, (1, nb), 1)
        iota_col = lax.broadcasted_iota(jnp.int32, (nb, 1), 0)
        mnb_b = bc_ref[...] == iota_row
        mnb = mnb_b.astype(_f32)
        emax = jnp.max(jnp.where(mnb_b, e_col, -1e30), axis=0, keepdims=True)
        emax_g = jnp.sum(mnb * emax, axis=1, keepdims=True)
        ex = jnp.exp(e_col - emax_g)
        esum = jnp.sum(jnp.where(mnb_b, ex, 0.0), axis=0, keepdims=True)
        esum_g = jnp.sum(mnb * esum, axis=1, keepdims=True)
        a_col = ex / esum_g
        xw = a_col * hv
        mbn = (iota_col == br_ref[...]).astype(_f32)
        r_read = jnp.dot(mbn, xw, preferred_element_type=_f32)
        out_ref[...] = jnp.concatenate(
            [jnp.broadcast_to(qv, (nb, 32)), r_read], axis=1)

    return pl.pallas_call(
        body,
        out_shape=jax.ShapeDtypeStruct((nb, 64), _f32),
    )(h, batch_col, batch_row, lstm_bias_row)


def kernel(x, edge_index, edge_attr, batch, lin0_W, lin0_b, net1_W, net1_b,
           net2_W, net2_b, root, conv_b, gru_Wih, gru_Whh, gru_bih, gru_bhh,
           lstm_Wih, lstm_Whh, lstm_bih, lstm_bhh):
    n, _ = x.shape
    e = edge_index.shape[1]
    nb = lstm_Whh.shape[1] * 2

    cpw = -(-e // (_NW * _CHUNK))
    e_pad = _NW * _CHUNK * cpw
    pad = e_pad - e

    src_p = jnp.concatenate([edge_index[0], jnp.zeros((pad,), jnp.int32)])
    dst_p = jnp.concatenate([edge_index[1], jnp.full((pad,), n, jnp.int32)])
    ea_p = jnp.concatenate([edge_attr, jnp.zeros((pad, 4), _f32)], axis=0)
    ones_p = jnp.concatenate([jnp.ones((e,), _f32), jnp.zeros((pad,), _f32)])
    zrows = jnp.zeros((_RPT, 32), _f32)
    zvec = jnp.zeros((_RPT,), _f32)

    degp = _sc_degree(dst_p, ones_p, zvec, cpw)
    deg0 = degp[:n].reshape(n, 1)
    deg1 = degp[_NPAD:_NPAD + n].reshape(n, 1)

    h, rdeg = _tc_prologue(x, lin0_W.T, lin0_b.reshape(1, 32), deg0, deg1)

    ea_aug_t = jnp.concatenate([ea_p.T, jnp.ones((1, e_pad), _f32)], axis=0)
    net1_aug = jnp.concatenate([
        jnp.concatenate([net1_W, net1_b.reshape(32, 1)], axis=1),
        jnp.array([[0.0, 0.0, 0.0, 0.0, 1.0]], _f32)], axis=0)
    net2_aug = jnp.concatenate([net2_W, net2_b.reshape(1024, 1)], axis=1)
    conv_b_row = conv_b.reshape(1, 32)
    wih_t = gru_Wih.T
    whh_t = gru_Whh.T
    bih_row = gru_bih.reshape(1, 96)
    bhh_row = gru_bhh.reshape(1, 96)

    e_act = _tc_edge_act(ea_aug_t, net1_aug)

    for _ in range(6):
        g = _sc_gather(h, src_p, cpw)
        msg = _tc_messages(g, e_act, net2_aug)
        parts = _sc_scatter(msg, dst_p, zrows, cpw)
        p0 = parts[:n]
        p1 = parts[_NPAD:_NPAD + n]
        h = _tc_gru(p0, p1, rdeg, h, root, conv_b_row, wih_t, whh_t,
                    bih_row, bhh_row)

    lstm_bias_row = (lstm_bih + lstm_bhh).reshape(1, 128)
    return _tc_set2set(h, batch.reshape(n, 1), batch.reshape(1, n),
                       lstm_bias_row, nb)
